# baseline (speedup 1.0000x reference)
import functools

import jax
import jax.numpy as jnp
from jax import lax
from jax.experimental import pallas as pl
from jax.experimental.pallas import tpu as pltpu
from jax.experimental.pallas import tpu_sc as plsc

H = 16
U = 729
N1 = 196
NP = 200
QROWS = 4
QP = QROWS * N1
QV = QP // 16
NQ = N1 // QROWS
L = 16
NW = 32


_mesh = plsc.VectorSubcoreMesh(core_axis_name="c", subcore_axis_name="s")


@functools.partial(
    pl.kernel,
    mesh=_mesh,
    compiler_params=pltpu.CompilerParams(
        needs_layout_passes=False, use_tc_tiling_on_sc=False
    ),
    out_type=jax.ShapeDtypeStruct((H, N1, N1), jnp.float32),
    scratch_types=[
        pltpu.VMEM((H * U,), jnp.float32),
        pltpu.VMEM((QP,), jnp.int32),
        pltpu.VMEM((H * QROWS, N1), jnp.float32),
        pltpu.SemaphoreType.DMA,
        pltpu.SemaphoreType.DMA,
        pltpu.SemaphoreType.DMA,
    ],
)
def _gather_bias(bias_hbm, idx_hbm, out_3d, bias_v, idx_v, out_v,
                 sem_b, sem_i, sem_o):
    cid = lax.axis_index("c")
    sid = lax.axis_index("s")
    wid = sid * 2 + cid

    cp_b = pltpu.async_copy(bias_hbm, bias_v, sem_b)
    lane = jnp.arange(L, dtype=jnp.int32)

    def do_quad(rq, first):
        cp_i = pltpu.async_copy(idx_hbm.at[pl.ds(rq * QP, QP)], idx_v, sem_i)
        cp_i.wait()
        if first:
            cp_b.wait()

        def step(v, carry):
            p = v * L + lane
            rv = p // N1
            cv = p - rv * N1
            iv = idx_v[pl.ds(v * L, L)]
            for h in range(H):
                vals = plsc.load_gather(bias_v, [iv + h * U])
                plsc.store_scatter(out_v, [rv + QROWS * h, cv], vals)
            return carry

        lax.fori_loop(0, QV, step, 0)
        cps = []
        for h in range(H):
            cps.append(pltpu.async_copy(
                out_v.at[pl.ds(QROWS * h, QROWS), :],
                out_3d.at[h, pl.ds(rq * QROWS, QROWS), :],
                sem_o))
        for cp in cps:
            cp.wait()

    do_quad(wid, True)

    @pl.when(wid < NQ - NW)
    def _():
        do_quad(wid + NW, False)


def kernel(bias, index):
    return _gather_bias(bias.reshape(H * U), index)
